# baseline (speedup 1.0000x reference)
import functools

import jax
import jax.numpy as jnp
from jax import lax
from jax.experimental import pallas as pl
from jax.experimental.pallas import tpu as pltpu
from jax.experimental.pallas import tpu_sc as plsc

B = 16384
D = 64
NCAT = 100000

_info = plsc.get_sparse_core_info()
_NC, _NS = _info.num_cores, _info.num_subcores
_NW = _NC * _NS
_BPW = B // _NW
_CHUNK = 128
_NCHUNK = _BPW // _CHUNK

_MESH = plsc.VectorSubcoreMesh(core_axis_name="c", subcore_axis_name="s")
_NO_TC_TILING = pltpu.CompilerParams(use_tc_tiling_on_sc=False)

_BLKR = 4000


def _relayout_body(x_ref, o_ref):
    o_ref[...] = jnp.concatenate(
        [x_ref[...], jnp.zeros((_BLKR, D), jnp.float32)], axis=1)


@jax.jit
def _relayout(cat_table):
    return pl.pallas_call(
        _relayout_body,
        grid=(NCAT // _BLKR,),
        in_specs=[pl.BlockSpec((_BLKR, D), lambda i: (i, 0))],
        out_specs=pl.BlockSpec((_BLKR, 2 * D), lambda i: (i, 0)),
        out_shape=jax.ShapeDtypeStruct((NCAT, 2 * D), jnp.float32),
    )(cat_table)


def _gather_cf_body(col_i, fab_i, colp, fabp, o_col, o_fab,
                    iv1, iv2, rv1, rv2, sem1, sem2):
    wid = lax.axis_index("s") * _NC + lax.axis_index("c")
    rbase = wid * _NCHUNK
    base = wid * _BPW
    half = _BPW // 2
    nh = _NCHUNK // 2
    pltpu.sync_copy(col_i.at[pl.ds(rbase, _NCHUNK)], iv1)
    pltpu.sync_copy(fab_i.at[pl.ds(rbase, _NCHUNK)], iv2)

    def fire(tab, iv, rv, sem, pass_i):
        return [
            pltpu.async_copy(tab.at[iv.at[pass_i * nh + j]],
                             rv.at[pl.ds(j * _CHUNK, _CHUNK)], sem)
            for j in range(nh)
        ]

    col_p = fire(colp, iv1, rv1, sem1, 0)
    fab_p = fire(fabp, iv2, rv2, sem2, 0)
    for c in col_p:
        c.wait()
    pltpu.sync_copy(rv1, o_col.at[pl.ds(base, half)])
    col_p = fire(colp, iv1, rv1, sem1, 1)
    for c in fab_p:
        c.wait()
    pltpu.sync_copy(rv2, o_fab.at[pl.ds(base, half)])
    fab_p = fire(fabp, iv2, rv2, sem2, 1)
    for c in col_p:
        c.wait()
    pltpu.sync_copy(rv1, o_col.at[pl.ds(base + half, half)])
    for c in fab_p:
        c.wait()
    pltpu.sync_copy(rv2, o_fab.at[pl.ds(base + half, half)])


def _gather_cat_body(cat_i, catlin, o_cat, iv0, rv0, sem0):
    wid = lax.axis_index("s") * _NC + lax.axis_index("c")
    rbase = wid * _NCHUNK
    base = wid * _BPW
    pltpu.sync_copy(cat_i.at[pl.ds(rbase, _NCHUNK)], iv0)
    copies = [
        pltpu.async_copy(catlin.at[iv0.at[j]],
                         rv0.at[pl.ds(j * _CHUNK, _CHUNK)], sem0)
        for j in range(_NCHUNK)
    ]
    for c in copies:
        c.wait()
    pltpu.sync_copy(rv0, o_cat.at[pl.ds(base, _BPW)])


@jax.jit
def _gather_all(cat2, col2, fab2, catlin, colp, fabp):
    f_cf = functools.partial(
        pl.kernel,
        mesh=_MESH,
        out_type=[jax.ShapeDtypeStruct((B, 2 * D), jnp.float32)] * 2,
        scratch_types=[pltpu.VMEM((_NCHUNK, _CHUNK), jnp.int32)] * 2
        + [pltpu.VMEM((_BPW // 2, 2 * D), jnp.float32)] * 2
        + [pltpu.SemaphoreType.DMA] * 2,
        compiler_params=_NO_TC_TILING,
    )(_gather_cf_body)
    f_cat = functools.partial(
        pl.kernel,
        mesh=_MESH,
        out_type=jax.ShapeDtypeStruct((B, 2 * D), jnp.float32),
        scratch_types=[pltpu.VMEM((_NCHUNK, _CHUNK), jnp.int32),
                       pltpu.VMEM((_BPW, 2 * D), jnp.float32),
                       pltpu.SemaphoreType.DMA],
        compiler_params=_NO_TC_TILING,
    )(_gather_cat_body)
    o_col, o_fab = f_cf(col2, fab2, colp, fabp)
    o_cat = f_cat(cat2, catlin)
    return o_cat, o_col, o_fab


def _fuse_body(x0_ref, x1_ref, x2_ref, wt_ref, b_ref, o_ref):
    wt = wt_ref[...]
    acc = jnp.dot(x0_ref[:, :D], wt[0:D, :], preferred_element_type=jnp.float32)
    acc += jnp.dot(x1_ref[:, :D], wt[D:2 * D, :], preferred_element_type=jnp.float32)
    acc += jnp.dot(x2_ref[:, :D], wt[2 * D:3 * D, :], preferred_element_type=jnp.float32)
    o_ref[...] = acc + b_ref[...]


_BLK = 2048


@jax.jit
def _fuse(x0, x1, x2, wt, b2):
    grid = (B // _BLK,)
    return pl.pallas_call(
        _fuse_body,
        grid=grid,
        in_specs=[
            pl.BlockSpec((_BLK, 2 * D), lambda i: (i, 0)),
            pl.BlockSpec((_BLK, 2 * D), lambda i: (i, 0)),
            pl.BlockSpec((_BLK, 2 * D), lambda i: (i, 0)),
            pl.BlockSpec((3 * D, D), lambda i: (0, 0)),
            pl.BlockSpec((1, D), lambda i: (0, 0)),
        ],
        out_specs=pl.BlockSpec((_BLK, D), lambda i: (i, 0)),
        out_shape=jax.ShapeDtypeStruct((B, D), jnp.float32),
    )(x0, x1, x2, wt, b2)


def kernel(cat, col, fab, cat_table, col_table, fab_table, W, b):
    catlin = _relayout(cat_table)
    colp = jnp.pad(col_table, ((0, 0), (0, D)))
    fabp = jnp.pad(fab_table, ((0, 0), (0, D)))
    cat2 = cat.astype(jnp.int32).reshape(B // _CHUNK, _CHUNK)
    col2 = col.astype(jnp.int32).reshape(B // _CHUNK, _CHUNK)
    fab2 = fab.astype(jnp.int32).reshape(B // _CHUNK, _CHUNK)
    x0, x1, x2 = _gather_all(cat2, col2, fab2, catlin, colp, fabp)
    return _fuse(x0, x1, x2, W.T, b.reshape(1, D))


# speedup vs baseline: 1.2504x; 1.2504x over previous
import functools

import jax
import jax.numpy as jnp
from jax import lax
from jax.experimental import pallas as pl
from jax.experimental.pallas import tpu as pltpu
from jax.experimental.pallas import tpu_sc as plsc

B = 16384
D = 64

_info = plsc.get_sparse_core_info()
_NC, _NS = _info.num_cores, _info.num_subcores
_NW = _NC * _NS
_BPW = B // _NW
_CHUNK = 128
_NCHUNK = _BPW // _CHUNK

_MESH = plsc.VectorSubcoreMesh(core_axis_name="c", subcore_axis_name="s")
_NO_TC_TILING = pltpu.CompilerParams(use_tc_tiling_on_sc=False)


def _gather_cf_body(col_i, fab_i, col_t, fab_t, o1,
                    iv1, iv2, rv1, rv2, sem1, sem2):
    wid = lax.axis_index("s") * _NC + lax.axis_index("c")
    base = wid * _BPW
    pltpu.sync_copy(col_i.at[pl.ds(base, _BPW)], iv1)
    pltpu.sync_copy(fab_i.at[pl.ds(base, _BPW)], iv2)
    copies = []
    for iv, tab, rv, sem in ((iv1, col_t, rv1, sem1),
                             (iv2, fab_t, rv2, sem2)):
        for j in range(_NCHUNK):
            sl = pl.ds(j * _CHUNK, _CHUNK)
            copies.append(pltpu.async_copy(tab.at[iv.at[sl]], rv.at[sl], sem))
    for c in copies:
        c.wait()
    rows = pl.ds(base, _BPW)
    pltpu.sync_copy(rv1, o1.at[rows, pl.ds(0, D)])
    pltpu.sync_copy(rv2, o1.at[rows, pl.ds(D, D)])


def _gather_cat_body(cat_i, cat_t, o2, iv0, rv0, sem0):
    wid = lax.axis_index("s") * _NC + lax.axis_index("c")
    base = wid * _BPW
    pltpu.sync_copy(cat_i.at[pl.ds(base, _BPW)], iv0)
    copies = []
    for j in range(_NCHUNK):
        sl = pl.ds(j * _CHUNK, _CHUNK)
        copies.append(pltpu.async_copy(cat_t.at[iv0.at[sl]], rv0.at[sl], sem0))
    for c in copies:
        c.wait()
    pltpu.sync_copy(rv0, o2.at[pl.ds(base, _BPW), pl.ds(0, D)])


@jax.jit
def _gather_all(cat, col, fab, cat_table, col_table, fab_table):
    f_cf = functools.partial(
        pl.kernel,
        mesh=_MESH,
        out_type=jax.ShapeDtypeStruct((B, 2 * D), jnp.float32),
        scratch_types=[pltpu.VMEM((_BPW,), jnp.int32)] * 2
        + [pltpu.VMEM((_BPW, D), jnp.float32)] * 2
        + [pltpu.SemaphoreType.DMA] * 2,
        compiler_params=_NO_TC_TILING,
    )(_gather_cf_body)
    f_cat = functools.partial(
        pl.kernel,
        mesh=_MESH,
        out_type=jax.ShapeDtypeStruct((B, 2 * D), jnp.float32),
        scratch_types=[pltpu.VMEM((_BPW,), jnp.int32),
                       pltpu.VMEM((_BPW, D), jnp.float32),
                       pltpu.SemaphoreType.DMA],
        compiler_params=_NO_TC_TILING,
    )(_gather_cat_body)
    x1 = f_cf(col, fab, col_table, fab_table)
    x2 = f_cat(cat, cat_table)
    return x1, x2


def _fuse_body(x1_ref, x2_ref, wt_ref, b_ref, o_ref):
    wt = wt_ref[...]
    acc = jnp.dot(x2_ref[:, :D], wt[0:D, :], preferred_element_type=jnp.float32)
    acc += jnp.dot(x1_ref[:, :D], wt[D:2 * D, :], preferred_element_type=jnp.float32)
    acc += jnp.dot(x1_ref[:, D:], wt[2 * D:3 * D, :], preferred_element_type=jnp.float32)
    o_ref[...] = acc + b_ref[...]


_BLK = 2048


@jax.jit
def _fuse(x1, x2, wt, b2):
    grid = (B // _BLK,)
    return pl.pallas_call(
        _fuse_body,
        grid=grid,
        in_specs=[
            pl.BlockSpec((_BLK, 2 * D), lambda i: (i, 0)),
            pl.BlockSpec((_BLK, 2 * D), lambda i: (i, 0)),
            pl.BlockSpec((3 * D, D), lambda i: (0, 0)),
            pl.BlockSpec((1, D), lambda i: (0, 0)),
        ],
        out_specs=pl.BlockSpec((_BLK, D), lambda i: (i, 0)),
        out_shape=jax.ShapeDtypeStruct((B, D), jnp.float32),
    )(x1, x2, wt, b2)


def kernel(cat, col, fab, cat_table, col_table, fab_table, W, b):
    x1, x2 = _gather_all(
        cat.astype(jnp.int32), col.astype(jnp.int32), fab.astype(jnp.int32),
        cat_table, col_table, fab_table)
    return _fuse(x1, x2, W.T, b.reshape(1, D))
